# baseline (speedup 1.0000x reference)
import functools

import jax
import jax.numpy as jnp
from jax import lax
from jax.experimental import pallas as pl
from jax.experimental.pallas import tpu as pltpu
from jax.experimental.pallas import tpu_sc as plsc

_N_NODES = 100000
_N_SEG = 2000
_SEG_PAD = 2048
_CHUNK = 6272
_CHUNK_LAST = _N_NODES - 15 * _CHUNK


def _seg_flags_body(batch_hbm, out_hbm, chunk, local, tmp, orow, shared):
    cid = lax.axis_index("c")
    sid = lax.axis_index("s")
    zeros16 = jnp.zeros((16,), jnp.float32)
    ones16 = jnp.ones((16,), jnp.float32)

    def _zero(j, c):
        local[pl.ds(j * 16, 16)] = zeros16
        return c

    lax.fori_loop(0, _SEG_PAD // 16, _zero, 0)

    @pl.when(sid < 15)
    def _():
        pltpu.sync_copy(batch_hbm.at[pl.ds(sid * _CHUNK, _CHUNK)], chunk)

    @pl.when(sid == 15)
    def _():
        pltpu.sync_copy(batch_hbm.at[pl.ds(15 * _CHUNK, _CHUNK_LAST)],
                        chunk.at[pl.ds(0, _CHUNK_LAST)])

    n_pair = jnp.where(sid == 15, _CHUNK_LAST // 32, _CHUNK // 32)

    def _scatter(i, c):
        idx0 = chunk[pl.ds(i * 32, 16)]
        idx1 = chunk[pl.ds(i * 32 + 16, 16)]
        plsc.store_scatter(local, [idx0], ones16)
        plsc.store_scatter(local, [idx1], ones16)
        return c

    lax.fori_loop(0, n_pair, _scatter, 0)

    pltpu.sync_copy(local, shared.at[sid])
    plsc.subcore_barrier()
    pltpu.sync_copy(shared.at[:, pl.ds(sid * 128, 128)], tmp)
    for j in range(8):
        acc = tmp[0, pl.ds(j * 16, 16)]
        for t in range(1, 16):
            acc = acc + tmp[t, pl.ds(j * 16, 16)]
        orow[pl.ds(j * 16, 16)] = jnp.where(acc > 0.0, 1.0, 0.0)

    @pl.when(cid == 0)
    def _():
        pltpu.sync_copy(orow, out_hbm.at[pl.ds(sid * 128, 128)])


_seg_flags = pl.kernel(
    _seg_flags_body,
    out_type=jax.ShapeDtypeStruct((_SEG_PAD,), jnp.float32),
    mesh=plsc.VectorSubcoreMesh(core_axis_name="c", subcore_axis_name="s",
                                num_cores=1),
    compiler_params=pltpu.CompilerParams(needs_layout_passes=False,
                                         skip_device_barrier=True),
    scratch_types=[
        pltpu.VMEM((_CHUNK,), jnp.int32),
        pltpu.VMEM((_SEG_PAD,), jnp.float32),
        pltpu.VMEM((16, 128), jnp.float32),
        pltpu.VMEM((128,), jnp.float32),
        pltpu.VMEM_SHARED((16, _SEG_PAD), jnp.float32),
    ],
    name="seg_nonempty_flags",
)


def _dot_t(a, b):
    return lax.dot_general(a, b, (((1,), (1,)), ((), ())),
                           preferred_element_type=jnp.float32)


def _pre_body(h_ref, tw0_ref, tb0_ref, whh0_ref, bhh0_ref, csp_ref, gh_ref):
    h = h_ref[...]
    csp_ref[...] = _dot_t(h, tw0_ref[...]) + tb0_ref[...][None, :]
    gh_ref[...] = _dot_t(h, whh0_ref[...]) + bhh0_ref[...][None, :]


def _gru_pre(h_s, tw0, tb0, whh0, bhh0):
    return pl.pallas_call(
        _pre_body,
        out_shape=(jax.ShapeDtypeStruct((_N_SEG, 128), jnp.float32),
                   jax.ShapeDtypeStruct((_N_SEG, 384), jnp.float32)),
    )(h_s, tw0, tb0, whh0, bhh0)


def _gru_body(h_ref, m_ref, csp_ref, gh0_ref,
              wih0_ref, bih0_ref,
              tw1_ref, tb1_ref, wih1_ref, whh1_ref, bih1_ref, bhh1_ref,
              out_ref):
    h = h_ref[...]
    m = m_ref[...].reshape(_SEG_PAD, 1)[0:_N_SEG, :]

    def gates(gi, gh, h):
        r = jax.nn.sigmoid(gi[:, 0:128] + gh[:, 0:128])
        z = jax.nn.sigmoid(gi[:, 128:256] + gh[:, 128:256])
        n = jnp.tanh(gi[:, 256:384] + r * gh[:, 256:384])
        return (1.0 - z) * n + z * h

    def elu(v):
        return jnp.where(v > 0.0, v, jnp.exp(jnp.minimum(v, 0.0)) - 1.0)

    cs = elu(csp_ref[...] * m)
    gi = _dot_t(cs, wih0_ref[...]) + bih0_ref[...][None, :]
    h = gates(gi, gh0_ref[...], h)
    cs = elu((_dot_t(h, tw1_ref[...]) + tb1_ref[...][None, :]) * m)
    gi = _dot_t(cs, wih1_ref[...]) + bih1_ref[...][None, :]
    gh = _dot_t(h, whh1_ref[...]) + bhh1_ref[...][None, :]
    h = gates(gi, gh, h)
    out_ref[...] = h


def _fused_gru(h_s, mask, csp, gh0, *weights):
    return pl.pallas_call(
        _gru_body,
        out_shape=jax.ShapeDtypeStruct((_N_SEG, 128), jnp.float32),
    )(h_s, mask, csp, gh0, *weights)


def kernel(h_s, x, batch, aw0, ab0, tw0, tb0, wih0, whh0, bih0, bhh0,
           aw1, ab1, tw1, tb1, wih1, whh1, bih1, bhh1):
    mask = _seg_flags(batch.astype(jnp.int32))
    csp, gh0 = _gru_pre(h_s, tw0, tb0, whh0, bhh0)
    return _fused_gru(
        h_s, mask, csp, gh0,
        wih0, bih0,
        tw1, tb1, wih1, whh1, bih1, bhh1,
    )


# speedup vs baseline: 1.0470x; 1.0470x over previous
import functools

import jax
import jax.numpy as jnp
from jax import lax
from jax.experimental import pallas as pl
from jax.experimental.pallas import tpu as pltpu
from jax.experimental.pallas import tpu_sc as plsc

_N_NODES = 100000
_N_SEG = 2000
_SEG_PAD = 2048
_CHUNK = 6272
_CHUNK_LAST = _N_NODES - 15 * _CHUNK


def _seg_flags_body(batch_hbm, out_hbm, chunk, local, tmp, orow, shared):
    cid = lax.axis_index("c")
    sid = lax.axis_index("s")
    zeros16 = jnp.zeros((16,), jnp.float32)
    ones16 = jnp.ones((16,), jnp.float32)

    def _zero(j, c):
        for u in range(4):
            local[pl.ds(j * 64 + u * 16, 16)] = zeros16
        return c

    lax.fori_loop(0, _SEG_PAD // 64, _zero, 0)

    @pl.when(sid < 15)
    def _():
        pltpu.sync_copy(batch_hbm.at[pl.ds(sid * _CHUNK, _CHUNK)], chunk)

    @pl.when(sid == 15)
    def _():
        pltpu.sync_copy(batch_hbm.at[pl.ds(15 * _CHUNK, _CHUNK_LAST)],
                        chunk.at[pl.ds(0, _CHUNK_LAST)])

    n_pair = jnp.where(sid == 15, _CHUNK_LAST // 32, _CHUNK // 32)

    def _scatter(i, c):
        idx0 = chunk[pl.ds(i * 32, 16)]
        idx1 = chunk[pl.ds(i * 32 + 16, 16)]
        plsc.store_scatter(local, [idx0], ones16)
        plsc.store_scatter(local, [idx1], ones16)
        return c

    lax.fori_loop(0, n_pair, _scatter, 0)

    pltpu.sync_copy(local, shared.at[sid])
    plsc.subcore_barrier()
    pltpu.sync_copy(shared.at[:, pl.ds(sid * 128, 128)], tmp)
    for j in range(8):
        acc = tmp[0, pl.ds(j * 16, 16)]
        for t in range(1, 16):
            acc = acc + tmp[t, pl.ds(j * 16, 16)]
        orow[pl.ds(j * 16, 16)] = jnp.where(acc > 0.0, 1.0, 0.0)

    @pl.when(cid == 0)
    def _():
        pltpu.sync_copy(orow, out_hbm.at[pl.ds(sid * 128, 128)])


_seg_flags = pl.kernel(
    _seg_flags_body,
    out_type=jax.ShapeDtypeStruct((_SEG_PAD,), jnp.float32),
    mesh=plsc.VectorSubcoreMesh(core_axis_name="c", subcore_axis_name="s",
                                num_cores=1),
    compiler_params=pltpu.CompilerParams(needs_layout_passes=False,
                                         skip_device_barrier=True),
    scratch_types=[
        pltpu.VMEM((_CHUNK,), jnp.int32),
        pltpu.VMEM((_SEG_PAD,), jnp.float32),
        pltpu.VMEM((16, 128), jnp.float32),
        pltpu.VMEM((128,), jnp.float32),
        pltpu.VMEM_SHARED((16, _SEG_PAD), jnp.float32),
    ],
    name="seg_nonempty_flags",
)


def _dot_t(a, b):
    return lax.dot_general(a, b, (((1,), (1,)), ((), ())),
                           preferred_element_type=jnp.float32)


def _gru_body(h_ref, m_ref,
              tw0_ref, tb0_ref, wih0_ref, whh0_ref, bih0_ref, bhh0_ref,
              tw1_ref, tb1_ref, wih1_ref, whh1_ref, bih1_ref, bhh1_ref,
              out_ref):
    h = h_ref[...]
    m = m_ref[...].reshape(_SEG_PAD, 1)[0:_N_SEG, :]

    def step(h, tw, tb, wih, whh, bih, bhh):
        cs = (_dot_t(h, tw) + tb[None, :]) * m
        cs = jnp.where(cs > 0.0, cs, jnp.exp(jnp.minimum(cs, 0.0)) - 1.0)
        gi = _dot_t(cs, wih) + bih[None, :]
        gh = _dot_t(h, whh) + bhh[None, :]
        r = jax.nn.sigmoid(gi[:, 0:128] + gh[:, 0:128])
        z = jax.nn.sigmoid(gi[:, 128:256] + gh[:, 128:256])
        n = jnp.tanh(gi[:, 256:384] + r * gh[:, 256:384])
        return (1.0 - z) * n + z * h

    h = step(h, tw0_ref[...], tb0_ref[...], wih0_ref[...], whh0_ref[...],
             bih0_ref[...], bhh0_ref[...])
    h = step(h, tw1_ref[...], tb1_ref[...], wih1_ref[...], whh1_ref[...],
             bih1_ref[...], bhh1_ref[...])
    out_ref[...] = h


def _fused_gru(h_s, mask, *weights):
    return pl.pallas_call(
        _gru_body,
        out_shape=jax.ShapeDtypeStruct((_N_SEG, 128), jnp.float32),
    )(h_s, mask, *weights)


def kernel(h_s, x, batch, aw0, ab0, tw0, tb0, wih0, whh0, bih0, bhh0,
           aw1, ab1, tw1, tb1, wih1, whh1, bih1, bhh1):
    mask = _seg_flags(batch.astype(jnp.int32))
    return _fused_gru(
        h_s, mask,
        tw0, tb0, wih0, whh0, bih0, bhh0,
        tw1, tb1, wih1, whh1, bih1, bhh1,
    )
